# baseline (speedup 1.0000x reference)
import jax
import jax.numpy as jnp
from jax.experimental import pallas as pl


def _add_body(x_ref, e_ref, o_ref):
    o_ref[...] = x_ref[...] + e_ref[...][None]


def kernel(x, emb_table):
    B, T, D = x.shape
    BT = 2048
    return pl.pallas_call(
        _add_body,
        grid=(T // BT, B),
        in_specs=[
            pl.BlockSpec((1, BT, D), lambda i, j: (j, i, 0)),
            pl.BlockSpec((BT, D), lambda i, j: (i, 0)),
        ],
        out_specs=pl.BlockSpec((1, BT, D), lambda i, j: (j, i, 0)),
        out_shape=jax.ShapeDtypeStruct(x.shape, x.dtype),
    )(x, emb_table)


# speedup vs baseline: 1.0004x; 1.0004x over previous
import jax
from jax.experimental import pallas as pl


def _add_body(x_ref, e_ref, o_ref):
    o_ref[...] = x_ref[...] + e_ref[...][None]


def kernel(x, emb_table):
    B, T, D = x.shape
    BT = 2048
    return pl.pallas_call(
        _add_body,
        grid=(T // BT, B),
        in_specs=[
            pl.BlockSpec((1, BT, D), lambda i, j: (j, i, 0)),
            pl.BlockSpec((BT, D), lambda i, j: (i, 0)),
        ],
        out_specs=pl.BlockSpec((1, BT, D), lambda i, j: (j, i, 0)),
        out_shape=jax.ShapeDtypeStruct(x.shape, x.dtype),
    )(x, emb_table)
